# Initial kernel scaffold; baseline (speedup 1.0000x reference)
import jax
import jax.numpy as jnp
from jax.experimental import pallas as pl


def kernel(feat_seq, para, W1, b1, W2, b2, Wp, bp, emb_table, Ws1, bs1, Ws2, bs2, gumbel):
    raise NotImplementedError("write your pallas kernel here")


# speedup vs baseline: 2.1611x; 2.1611x over previous
import dataclasses
import functools

import jax
import jax.numpy as jnp
from jax import lax
from jax.experimental import pallas as pl
from jax.experimental.pallas import tpu as pltpu
from jax.experimental.pallas import tpu_sc as plsc

B = 32
HID = 64
TP = 8192
KMID = 30
NEG = -3.0e38


_CDIM = 136


def _scores_body(embT_ref, gum_ref, feat_ref, para_ref, W1_ref, b1_ref,
                 W2_ref, b2_ref, Wp_ref, bp_ref, Ws1_ref, bs1_ref, ws2_ref,
                 y_ref, fpT_s, c_s, wcat_s, xa_s):
    b = pl.program_id(0)

    @pl.when(b == 0)
    def _init():
        h1 = jnp.maximum(
            jax.lax.dot_general(para_ref[...], W1_ref[...],
                                (((1,), (0,)), ((), ())),
                                preferred_element_type=jnp.float32,
                                precision=jax.lax.Precision.HIGHEST)
            + b1_ref[...], 0.0)
        para_emb = jax.lax.dot_general(h1, W2_ref[...],
                                       (((1,), (0,)), ((), ())),
                                       preferred_element_type=jnp.float32,
                                       precision=jax.lax.Precision.HIGHEST) \
            + b2_ref[...]
        feat_cat = jnp.concatenate([feat_ref[...], para_emb], axis=1)
        fpT = jax.lax.dot_general(Wp_ref[...], feat_cat,
                                  (((0,), (1,)), ((), ())),
                                  preferred_element_type=jnp.float32,
                                  precision=jax.lax.Precision.HIGHEST) \
            + bp_ref[...]
        fpT_s[...] = fpT
        a13 = Ws1_ref[0:64, :] + Ws1_ref[128:192, :]
        c_s[...] = jax.lax.dot_general(fpT, a13, (((0,), (0,)), ((), ())),
                                       preferred_element_type=jnp.float32,
                                       precision=jax.lax.Precision.HIGHEST) \
            + bs1_ref[...]
        wcat_s[0:64, :] = Ws1_ref[192:256, :]
        wcat_s[64:128, :] = Ws1_ref[64:128, :] - Ws1_ref[128:192, :]
        wcat_s[128:136, :] = jnp.zeros((8, HID), jnp.float32)
        xa_s[64:128, :] = embT_ref[...]
        xa_s[128:129, :] = jnp.ones((1, TP), jnp.float32)
        xa_s[129:136, :] = jnp.zeros((7, TP), jnp.float32)

    bsel_r = lax.broadcasted_iota(jnp.int32, (B, HID), 0) == b
    wcat_s[128:129, :] = jnp.sum(jnp.where(bsel_r, c_s[...], 0.0), axis=0,
                                 keepdims=True)
    bsel_c = lax.broadcasted_iota(jnp.int32, (HID, B), 1) == b
    q_col = jnp.sum(jnp.where(bsel_c, fpT_s[...], 0.0), axis=1,
                    keepdims=True)
    xa_s[0:64, :] = embT_ref[...] * q_col
    hT = jax.lax.dot_general(wcat_s[...], xa_s[...], (((0,), (0,)), ((), ())),
                             preferred_element_type=jnp.float32,
                             precision=jax.lax.Precision.HIGHEST)
    rh = jnp.maximum(hT, 0.0)
    s = jax.lax.dot_general(ws2_ref[...], rh, (((0,), (0,)), ((), ())),
                            preferred_element_type=jnp.float32,
                            precision=jax.lax.Precision.HIGHEST)
    y_ref[0] = s + gum_ref[0]


def _scores(embT, gum, feat_seq, para, W1, b1, W2, b2, Wp, bp, Ws1, bs1, ws2):
    full = lambda shape: pl.BlockSpec(shape, lambda b: (0, 0))
    return pl.pallas_call(
        _scores_body,
        grid=(B,),
        in_specs=[
            full((HID, TP)),
            pl.BlockSpec((1, 1, TP), lambda b: (b, 0, 0)),
            full((B, 1024)),
            full((B, 2)),
            full((2, 2 * HID)),
            full((1, 2 * HID)),
            full((2 * HID, HID)),
            full((1, HID)),
            full((1024 + HID, HID)),
            full((HID, 1)),
            full((4 * HID, HID)),
            full((1, HID)),
            full((HID, 1)),
        ],
        out_specs=pl.BlockSpec((1, 1, TP), lambda b: (b, 0, 0)),
        out_shape=jax.ShapeDtypeStruct((B, 1, TP), jnp.float32),
        scratch_shapes=[
            pltpu.VMEM((HID, B), jnp.float32),
            pltpu.VMEM((B, HID), jnp.float32),
            pltpu.VMEM((_CDIM, HID), jnp.float32),
            pltpu.VMEM((_CDIM, TP), jnp.float32),
        ],
        compiler_params=pltpu.CompilerParams(
            dimension_semantics=("arbitrary",)),
    )(embT, gum, feat_seq, para, W1, b1, W2, b2, Wp, bp, Ws1, bs1, ws2)


_NGRP = 32
_BIG = 1 << 30


def _topk_body(y_hbm, out_hbm, row_v, p_v, idx_v, sem):
    wid = lax.axis_index("s") * 2 + lax.axis_index("c")
    pltpu.async_copy(y_hbm.at[wid], row_v, sem).wait()
    iota16 = lax.iota(jnp.int32, 16)

    @pl.loop(0, _NGRP)
    def _build(g):
        base = g * 256
        m = row_v[pl.ds(base, 16)]
        for i in range(1, 16):
            m = jnp.maximum(m, row_v[pl.ds(base + i * 16, 16)])
        p_v[pl.ds(g * 16, 16)] = m

    @pl.loop(0, KMID)
    def _select(k):
        m16 = p_v[pl.ds(0, 16)]
        for g in range(1, _NGRP):
            m16 = jnp.maximum(m16, p_v[pl.ds(g * 16, 16)])
        gm = jnp.max(m16)
        l = jnp.min(jnp.where(m16 == gm, iota16, _BIG))
        gidx = iota16 * 16 + l
        g1 = plsc.load_gather(p_v, [gidx])
        g2 = plsc.load_gather(p_v, [gidx + 256])
        j1 = jnp.min(jnp.where(g1 == gm, iota16, _BIG))
        j2 = jnp.min(jnp.where(g2 == gm, iota16 + 16, _BIG))
        jst = jnp.minimum(j1, j2)
        ridx = jst * 256 + iota16 * 16 + l
        rv = plsc.load_gather(row_v, [ridx])
        rst = jnp.min(jnp.where(rv == gm, iota16, _BIG))
        t = jst * 256 + rst * 16 + l
        slot = k + 1
        v0 = idx_v[pl.ds(0, 16)]
        idx_v[pl.ds(0, 16)] = jnp.where(iota16 == slot, t + 1, v0)
        v1 = idx_v[pl.ds(16, 16)]
        idx_v[pl.ds(16, 16)] = jnp.where(iota16 + 16 == slot, t + 1, v1)
        vbase = jst * 256 + rst * 16
        vec = row_v[pl.ds(vbase, 16)]
        row_v[pl.ds(vbase, 16)] = jnp.where(iota16 == l, NEG, vec)
        m = row_v[pl.ds(jst * 256, 16)]
        for i in range(1, 16):
            m = jnp.maximum(m, row_v[pl.ds(jst * 256 + i * 16, 16)])
        p_v[pl.ds(jst * 16, 16)] = m

    v0 = jnp.where(iota16 == 0, 0, idx_v[pl.ds(0, 16)])
    v1 = jnp.where(iota16 == 15, TP - 1, idx_v[pl.ds(16, 16)])
    sa = lax.sort(v0, dimension=0)
    sb = lax.rev(lax.sort(v1, dimension=0), (0,))
    lo = jnp.minimum(sa, sb)
    hi = jnp.maximum(sa, sb)
    idx_v[pl.ds(0, 16)] = lax.sort(lo, dimension=0)
    idx_v[pl.ds(16, 16)] = lax.sort(hi, dimension=0)
    pltpu.sync_copy(idx_v, out_hbm.at[wid])


def _topk(y):
    mesh = plsc.VectorSubcoreMesh(core_axis_name="c", subcore_axis_name="s")
    cp = pltpu.CompilerParams()
    if "needs_layout_passes" in pltpu.CompilerParams.__dataclass_fields__:
        cp = dataclasses.replace(cp, needs_layout_passes=False)
    kern = functools.partial(
        pl.kernel,
        out_type=jax.ShapeDtypeStruct((B, B), jnp.int32),
        mesh=mesh,
        compiler_params=cp,
        scratch_types=[
            pltpu.VMEM((TP,), jnp.float32),
            pltpu.VMEM((_NGRP * 16,), jnp.float32),
            pltpu.VMEM((B,), jnp.int32),
            pltpu.SemaphoreType.DMA,
        ],
    )(_topk_body)
    return kern(y)


def kernel(feat_seq, para, W1, b1, W2, b2, Wp, bp, emb_table, Ws1, bs1, Ws2,
           bs2, gumbel):
    embT = jnp.pad(emb_table, ((0, 2), (0, 0))).T
    gum = jnp.pad(gumbel, ((0, 0), (0, 2)),
                  constant_values=-jnp.inf).reshape(B, 1, TP)
    y = _scores(embT, gum, feat_seq, para, W1, b1.reshape(1, 2 * HID), W2,
                b2.reshape(1, HID), Wp, bp.reshape(HID, 1), Ws1,
                bs1.reshape(1, HID), Ws2.reshape(HID, 1))
    return _topk(y.reshape(B, TP))


# speedup vs baseline: 5.1242x; 2.3711x over previous
import dataclasses
import functools

import jax
import jax.numpy as jnp
from jax import lax
from jax.experimental import pallas as pl
from jax.experimental.pallas import tpu as pltpu
from jax.experimental.pallas import tpu_sc as plsc

B = 32
HID = 64
TP = 8192
KMID = 30
NEG = -3.0e38


_RB = 4
_M = _RB * HID


def _scores_body(embT_ref, gum_ref, feat_ref, para_ref, W1_ref, b1_ref,
                 W2_ref, b2_ref, Wp_ref, bp_ref, Ws1_ref, bs1_ref, ws2_ref,
                 y_ref, fpT_s, cTt_s, a4t_s, a23t_s, ws2t_s, e4_s):
    i = pl.program_id(0)

    @pl.when(i == 0)
    def _init():
        h1 = jnp.maximum(
            jax.lax.dot_general(para_ref[...], W1_ref[...],
                                (((1,), (0,)), ((), ())),
                                preferred_element_type=jnp.float32,
                                precision=jax.lax.Precision.HIGHEST)
            + b1_ref[...], 0.0)
        para_emb = jax.lax.dot_general(h1, W2_ref[...],
                                       (((1,), (0,)), ((), ())),
                                       preferred_element_type=jnp.float32,
                                       precision=jax.lax.Precision.HIGHEST) \
            + b2_ref[...]
        feat_cat = jnp.concatenate([feat_ref[...], para_emb], axis=1)
        fpT = jax.lax.dot_general(Wp_ref[...], feat_cat,
                                  (((0,), (1,)), ((), ())),
                                  preferred_element_type=jnp.float32,
                                  precision=jax.lax.Precision.HIGHEST) \
            + bp_ref[...]
        fpT_s[...] = fpT
        a13 = Ws1_ref[0:64, :] + Ws1_ref[128:192, :]
        cT = jax.lax.dot_general(a13, fpT, (((0,), (0,)), ((), ())),
                                 preferred_element_type=jnp.float32,
                                 precision=jax.lax.Precision.HIGHEST) \
            + bs1_ref[...]
        cTt_s[...] = jnp.concatenate([cT] * _RB, axis=0)
        a4t_s[...] = jnp.concatenate([Ws1_ref[192:256, :]] * _RB, axis=1)
        a23 = Ws1_ref[64:128, :] - Ws1_ref[128:192, :]
        a23t_s[...] = jnp.concatenate([a23] * _RB, axis=1)
        ws2t_s[...] = jnp.concatenate([ws2_ref[...]] * _RB, axis=0)
        ehi = embT_ref[...].astype(jnp.bfloat16)
        elo = (embT_ref[...] - ehi.astype(jnp.float32)).astype(jnp.bfloat16)
        e4_s[...] = jnp.concatenate([ehi, elo, elo, ehi], axis=0)

    base = _RB * i
    sq = (lax.broadcasted_iota(jnp.int32, (B, _M), 0)
          == base + lax.broadcasted_iota(jnp.int32, (B, _M), 1) // HID
          ).astype(jnp.float32)
    q4 = jax.lax.dot_general(fpT_s[...], sq, (((1,), (0,)), ((), ())),
                             preferred_element_type=jnp.float32,
                             precision=jax.lax.Precision.HIGHEST)
    selc = (lax.broadcasted_iota(jnp.int32, (_M, B), 1)
            == base + lax.broadcasted_iota(jnp.int32, (_M, B), 0) // HID)
    c4 = jnp.sum(jnp.where(selc, cTt_s[...], 0.0), axis=1,
                 keepdims=True)
    ell = q4 * a4t_s[...] + a23t_s[...]
    lhi = ell.astype(jnp.bfloat16)
    llo = (ell - lhi.astype(jnp.float32)).astype(jnp.bfloat16)
    lst = jnp.concatenate([lhi, llo, lhi, llo], axis=0)
    hT4 = jax.lax.dot_general(lst, e4_s[...], (((0,), (0,)), ((), ())),
                              preferred_element_type=jnp.float32)
    mh = jnp.maximum(hT4 + c4, 0.0) * ws2t_s[...]
    s4 = jnp.concatenate(
        [jnp.sum(mh[HID * r:HID * (r + 1), :], axis=0, keepdims=True)
         for r in range(_RB)], axis=0)
    y_ref[0] = s4 + gum_ref[0]


def _scores(embT, gum, feat_seq, para, W1, b1, W2, b2, Wp, bp, Ws1, bs1, ws2):
    full = lambda shape: pl.BlockSpec(shape, lambda b: (0, 0))
    return pl.pallas_call(
        _scores_body,
        grid=(B // _RB,),
        in_specs=[
            full((HID, TP)),
            pl.BlockSpec((1, _RB, TP), lambda b: (b, 0, 0)),
            full((B, 1024)),
            full((B, 2)),
            full((2, 2 * HID)),
            full((1, 2 * HID)),
            full((2 * HID, HID)),
            full((1, HID)),
            full((1024 + HID, HID)),
            full((HID, 1)),
            full((4 * HID, HID)),
            full((HID, 1)),
            full((HID, 1)),
        ],
        out_specs=pl.BlockSpec((1, _RB, TP), lambda b: (b, 0, 0)),
        out_shape=jax.ShapeDtypeStruct((B // _RB, _RB, TP), jnp.float32),
        scratch_shapes=[
            pltpu.VMEM((HID, B), jnp.float32),
            pltpu.VMEM((_M, B), jnp.float32),
            pltpu.VMEM((HID, _M), jnp.float32),
            pltpu.VMEM((HID, _M), jnp.float32),
            pltpu.VMEM((_M, 1), jnp.float32),
            pltpu.VMEM((4 * HID, TP), jnp.bfloat16),
        ],
        compiler_params=pltpu.CompilerParams(
            dimension_semantics=("arbitrary",)),
    )(embT, gum, feat_seq, para, W1, b1, W2, b2, Wp, bp, Ws1, bs1, ws2)


_NGRP = 32
_BIG = 1 << 30


def _topk_body(y_hbm, out_hbm, row_v, p_v, idx_v, sem):
    wid = lax.axis_index("s") * 2 + lax.axis_index("c")
    pltpu.async_copy(y_hbm.at[wid], row_v, sem).wait()
    iota16 = lax.iota(jnp.int32, 16)

    @pl.loop(0, _NGRP)
    def _build(g):
        base = g * 256
        m = row_v[pl.ds(base, 16)]
        for i in range(1, 16):
            m = jnp.maximum(m, row_v[pl.ds(base + i * 16, 16)])
        p_v[pl.ds(g * 16, 16)] = m

    @pl.loop(0, KMID)
    def _select(k):
        m16 = p_v[pl.ds(0, 16)]
        for g in range(1, _NGRP):
            m16 = jnp.maximum(m16, p_v[pl.ds(g * 16, 16)])
        gm = jnp.max(m16)
        l = jnp.min(jnp.where(m16 == gm, iota16, _BIG))
        gidx = iota16 * 16 + l
        g1 = plsc.load_gather(p_v, [gidx])
        g2 = plsc.load_gather(p_v, [gidx + 256])
        j1 = jnp.min(jnp.where(g1 == gm, iota16, _BIG))
        j2 = jnp.min(jnp.where(g2 == gm, iota16 + 16, _BIG))
        jst = jnp.minimum(j1, j2)
        ridx = jst * 256 + iota16 * 16 + l
        rv = plsc.load_gather(row_v, [ridx])
        rst = jnp.min(jnp.where(rv == gm, iota16, _BIG))
        t = jst * 256 + rst * 16 + l
        slot = k + 1
        v0 = idx_v[pl.ds(0, 16)]
        idx_v[pl.ds(0, 16)] = jnp.where(iota16 == slot, t + 1, v0)
        v1 = idx_v[pl.ds(16, 16)]
        idx_v[pl.ds(16, 16)] = jnp.where(iota16 + 16 == slot, t + 1, v1)
        vbase = jst * 256 + rst * 16
        vec = row_v[pl.ds(vbase, 16)]
        row_v[pl.ds(vbase, 16)] = jnp.where(iota16 == l, NEG, vec)
        m = row_v[pl.ds(jst * 256, 16)]
        for i in range(1, 16):
            m = jnp.maximum(m, row_v[pl.ds(jst * 256 + i * 16, 16)])
        p_v[pl.ds(jst * 16, 16)] = m

    v0 = jnp.where(iota16 == 0, 0, idx_v[pl.ds(0, 16)])
    v1 = jnp.where(iota16 == 15, TP - 1, idx_v[pl.ds(16, 16)])
    sa = lax.sort(v0, dimension=0)
    sb = lax.rev(lax.sort(v1, dimension=0), (0,))
    lo = jnp.minimum(sa, sb)
    hi = jnp.maximum(sa, sb)
    idx_v[pl.ds(0, 16)] = lax.sort(lo, dimension=0)
    idx_v[pl.ds(16, 16)] = lax.sort(hi, dimension=0)
    pltpu.sync_copy(idx_v, out_hbm.at[wid])


def _topk(y):
    mesh = plsc.VectorSubcoreMesh(core_axis_name="c", subcore_axis_name="s")
    cp = pltpu.CompilerParams()
    if "needs_layout_passes" in pltpu.CompilerParams.__dataclass_fields__:
        cp = dataclasses.replace(cp, needs_layout_passes=False)
    kern = functools.partial(
        pl.kernel,
        out_type=jax.ShapeDtypeStruct((B, B), jnp.int32),
        mesh=mesh,
        compiler_params=cp,
        scratch_types=[
            pltpu.VMEM((TP,), jnp.float32),
            pltpu.VMEM((_NGRP * 16,), jnp.float32),
            pltpu.VMEM((B,), jnp.int32),
            pltpu.SemaphoreType.DMA,
        ],
    )(_topk_body)
    return kern(y)


def kernel(feat_seq, para, W1, b1, W2, b2, Wp, bp, emb_table, Ws1, bs1, Ws2,
           bs2, gumbel):
    embT = jnp.pad(emb_table, ((0, 2), (0, 0))).T
    gum = jnp.pad(gumbel, ((0, 0), (0, 2)),
                  constant_values=-jnp.inf).reshape(B // _RB, _RB, TP)
    y = _scores(embT, gum, feat_seq, para, W1, b1.reshape(1, 2 * HID), W2,
                b2.reshape(1, HID), Wp, bp.reshape(HID, 1), Ws1,
                bs1.reshape(HID, 1), Ws2.reshape(HID, 1))
    return _topk(y.reshape(B, TP))
